# Initial kernel scaffold; baseline (speedup 1.0000x reference)
import jax
import jax.numpy as jnp
from jax.experimental import pallas as pl


def kernel(x, edge_index):
    raise NotImplementedError("write your pallas kernel here")


# speedup vs baseline: 5.5266x; 5.5266x over previous
import functools

import jax
import jax.numpy as jnp
from jax import lax
from jax.experimental import pallas as pl
from jax.experimental.pallas import tpu as pltpu
from jax.experimental.pallas import tpu_sc as plsc

N_NODES = 10000
D = 128
N_EDGES = 320000

NC = 2
NS = 16
CHUNK = 80

EDGES_PER_SC = N_EDGES // NC
EDGES_PER_TILE = EDGES_PER_SC // NS
N_ITERS = EDGES_PER_TILE // CHUNK

ROWS_PER_TILE = 624


def _make_sc_kernel():
    mesh = plsc.VectorSubcoreMesh(core_axis_name="c", subcore_axis_name="s")

    @functools.partial(
        pl.kernel,
        out_type=jax.ShapeDtypeStruct((NC * N_NODES, D), jnp.float32),
        mesh=mesh,
        scratch_types=[
            pltpu.VMEM_SHARED((N_NODES, D), jnp.float32),
            pltpu.VMEM((CHUNK,), jnp.int32),
            pltpu.VMEM((CHUNK,), jnp.int32),
            pltpu.VMEM((CHUNK, D), jnp.float32),
            pltpu.VMEM((16, D), jnp.float32),
            pltpu.SemaphoreType.DMA,
        ],
    )
    def sc_kernel(x_hbm, src_hbm, dst_hbm, out_hbm, accum, sidx, didx, rows,
                  zbuf, sem):
        c = lax.axis_index("c")
        s = lax.axis_index("s")

        zero_v = jnp.zeros((16,), jnp.float32)
        for r in range(16):
            for j in range(D // 16):
                zbuf[r, pl.ds(j * 16, 16)] = zero_v

        row0 = s * ROWS_PER_TILE
        nz = jnp.where(s == NS - 1, 40, 39)

        @pl.loop(0, nz)
        def _zero(i):
            pltpu.sync_copy(zbuf, accum.at[pl.ds(row0 + i * 16, 16)])

        plsc.subcore_barrier()

        e0 = c * EDGES_PER_SC + s * EDGES_PER_TILE

        @pl.loop(0, N_ITERS)
        def _edges(i):
            e = e0 + i * CHUNK
            pltpu.sync_copy(src_hbm.at[pl.ds(e, CHUNK)], sidx)
            pltpu.sync_copy(dst_hbm.at[pl.ds(e, CHUNK)], didx)
            pltpu.async_copy(x_hbm.at[sidx], rows, sem).wait()
            pltpu.sync_copy(rows, accum.at[didx], add=True)

        plsc.subcore_barrier()

        out_row0 = c * N_NODES + row0
        pltpu.sync_copy(accum.at[pl.ds(row0, ROWS_PER_TILE)],
                        out_hbm.at[pl.ds(out_row0, ROWS_PER_TILE)])

        @pl.when(s == NS - 1)
        def _tail():
            pltpu.sync_copy(accum.at[pl.ds(NS * ROWS_PER_TILE, 16)],
                            out_hbm.at[pl.ds(c * N_NODES + NS * ROWS_PER_TILE,
                                             16)])

    return sc_kernel


_sc_kernel = _make_sc_kernel()


def _combine_body(a_ref, b_ref, o_ref):
    o_ref[...] = a_ref[...] + b_ref[...]


@jax.jit
def _combine(partials):
    blk = 1000
    grid = N_NODES // blk
    return pl.pallas_call(
        _combine_body,
        grid=(grid,),
        in_specs=[
            pl.BlockSpec((blk, D), lambda i: (i, 0)),
            pl.BlockSpec((blk, D), lambda i: (i + N_NODES // blk, 0)),
        ],
        out_specs=pl.BlockSpec((blk, D), lambda i: (i, 0)),
        out_shape=jax.ShapeDtypeStruct((N_NODES, D), jnp.float32),
    )(partials, partials)


@jax.jit
def kernel(x, edge_index):
    src = edge_index[0]
    dst = edge_index[1]
    partials = _sc_kernel(x, src, dst)
    return _combine(partials)


# speedup vs baseline: 8.2338x; 1.4898x over previous
import functools

import jax
import jax.numpy as jnp
from jax import lax
from jax.experimental import pallas as pl
from jax.experimental.pallas import tpu as pltpu
from jax.experimental.pallas import tpu_sc as plsc

N_NODES = 10000
D = 128
N_EDGES = 320000

NC = 2
NS = 16
CHUNK = 80

EDGES_PER_SC = N_EDGES // NC
EDGES_PER_TILE = EDGES_PER_SC // NS
N_ITERS = EDGES_PER_TILE // CHUNK

ROWS_PER_TILE = 624


def _make_sc_kernel():
    mesh = plsc.VectorSubcoreMesh(core_axis_name="c", subcore_axis_name="s")

    @functools.partial(
        pl.kernel,
        out_type=jax.ShapeDtypeStruct((NC * N_NODES, D), jnp.float32),
        mesh=mesh,
        scratch_types=[
            pltpu.VMEM_SHARED((N_NODES, D), jnp.float32),
            pltpu.VMEM((CHUNK,), jnp.int32),
            pltpu.VMEM((CHUNK,), jnp.int32),
            pltpu.VMEM((CHUNK,), jnp.int32),
            pltpu.VMEM((CHUNK,), jnp.int32),
            pltpu.VMEM((CHUNK, D), jnp.float32),
            pltpu.VMEM((CHUNK, D), jnp.float32),
            pltpu.VMEM((16, D), jnp.float32),
            pltpu.SemaphoreType.DMA,
            pltpu.SemaphoreType.DMA,
            pltpu.SemaphoreType.DMA,
            pltpu.SemaphoreType.DMA,
        ],
    )
    def sc_kernel(x_hbm, src_hbm, dst_hbm, out_hbm, accum, sidx0, sidx1,
                  didx0, didx1, rows0, rows1, zbuf, semg0, semg1, sems0,
                  sems1):
        c = lax.axis_index("c")
        s = lax.axis_index("s")
        w = c * NS + s

        zero_v = jnp.zeros((16,), jnp.float32)
        for r in range(16):
            for j in range(D // 16):
                zbuf[r, pl.ds(j * 16, 16)] = zero_v

        row0 = s * ROWS_PER_TILE
        nz = jnp.where(s == NS - 1, 40, 39)

        @pl.loop(0, nz)
        def _zero(i):
            pltpu.sync_copy(zbuf, accum.at[pl.ds(row0 + i * 16, 16)])

        plsc.subcore_barrier()

        def idxcopy(i, sidx, didx):
            pltpu.sync_copy(src_hbm.at[w, i], sidx)
            pltpu.sync_copy(dst_hbm.at[w, i], didx)

        def gather(sidx, rows, semg):
            return pltpu.async_copy(x_hbm.at[sidx], rows, semg)

        def scatter(didx, rows, sems):
            return pltpu.async_copy(rows, accum.at[didx], sems, add=True)

        def wait_gather(sidx, rows, semg):
            pltpu.make_async_copy(x_hbm.at[sidx], rows, semg).wait()

        def wait_scatter(didx, rows, sems):
            pltpu.make_async_copy(rows, accum.at[didx], sems).wait()

        idxcopy(0, sidx0, didx0)
        gather(sidx0, rows0, semg0)
        idxcopy(1, sidx1, didx1)
        gather(sidx1, rows1, semg1)
        wait_gather(sidx0, rows0, semg0)
        scatter(didx0, rows0, sems0)

        @pl.loop(0, (N_ITERS - 1) // 2)
        def _pairs(g):
            i = 2 * g + 1
            wait_scatter(didx0, rows0, sems0)
            idxcopy(i + 1, sidx0, didx0)
            gather(sidx0, rows0, semg0)
            wait_gather(sidx1, rows1, semg1)
            scatter(didx1, rows1, sems1)

            @pl.when(i + 2 < N_ITERS)
            def _more():
                wait_scatter(didx1, rows1, sems1)
                idxcopy(i + 2, sidx1, didx1)
                gather(sidx1, rows1, semg1)

            wait_gather(sidx0, rows0, semg0)
            scatter(didx0, rows0, sems0)

        wait_scatter(didx1, rows1, sems1)
        wait_scatter(didx0, rows0, sems0)

        plsc.subcore_barrier()

        out_row0 = c * N_NODES + row0
        pltpu.sync_copy(accum.at[pl.ds(row0, ROWS_PER_TILE)],
                        out_hbm.at[pl.ds(out_row0, ROWS_PER_TILE)])

        @pl.when(s == NS - 1)
        def _tail():
            pltpu.sync_copy(accum.at[pl.ds(NS * ROWS_PER_TILE, 16)],
                            out_hbm.at[pl.ds(c * N_NODES + NS * ROWS_PER_TILE,
                                             16)])

    return sc_kernel


_sc_kernel = _make_sc_kernel()


def _combine_body(a_ref, b_ref, o_ref):
    o_ref[...] = a_ref[...] + b_ref[...]


@jax.jit
def _combine(partials):
    blk = 1000
    grid = N_NODES // blk
    return pl.pallas_call(
        _combine_body,
        grid=(grid,),
        in_specs=[
            pl.BlockSpec((blk, D), lambda i: (i, 0)),
            pl.BlockSpec((blk, D), lambda i: (i + N_NODES // blk, 0)),
        ],
        out_specs=pl.BlockSpec((blk, D), lambda i: (i, 0)),
        out_shape=jax.ShapeDtypeStruct((N_NODES, D), jnp.float32),
    )(partials, partials)


@jax.jit
def kernel(x, edge_index):
    src = edge_index[0].reshape(NC * NS, N_ITERS, CHUNK)
    dst = edge_index[1].reshape(NC * NS, N_ITERS, CHUNK)
    partials = _sc_kernel(x, src, dst)
    return _combine(partials)


# speedup vs baseline: 12.1030x; 1.4699x over previous
import functools

import jax
import jax.numpy as jnp
from jax import lax
from jax.experimental import pallas as pl
from jax.experimental.pallas import tpu as pltpu
from jax.experimental.pallas import tpu_sc as plsc

N_NODES = 10000
D = 128
N_EDGES = 320000

NC = 2
NS = 16
CHUNK = 80

EDGES_PER_SC = N_EDGES // NC
EDGES_PER_TILE = EDGES_PER_SC // NS
N_ITERS = EDGES_PER_TILE // CHUNK

ROWS_PER_TILE = 624


def _make_sc_kernel():
    mesh = plsc.VectorSubcoreMesh(core_axis_name="c", subcore_axis_name="s")

    @functools.partial(
        pl.kernel,
        out_type=jax.ShapeDtypeStruct((NC * N_NODES, D), jnp.float32),
        mesh=mesh,
        scratch_types=[
            pltpu.VMEM_SHARED((N_NODES, D), jnp.float32),
            [pltpu.VMEM((2, CHUNK), jnp.int32) for _ in range(3)],
            [pltpu.VMEM((CHUNK, D), jnp.float32) for _ in range(3)],
            pltpu.VMEM((16, D), jnp.float32),
            [pltpu.SemaphoreType.DMA for _ in range(3)],
            [pltpu.SemaphoreType.DMA for _ in range(3)],
        ],
    )
    def sc_kernel(x_hbm, eidx_hbm, out_hbm, accum, ebuf, rows, zbuf,
                  semg, sems):
        c = lax.axis_index("c")
        s = lax.axis_index("s")
        w = c * NS + s

        zero_v = jnp.zeros((16,), jnp.float32)
        for r in range(16):
            for j in range(D // 16):
                zbuf[r, pl.ds(j * 16, 16)] = zero_v

        row0 = s * ROWS_PER_TILE
        nz = jnp.where(s == NS - 1, 40, 39)

        @pl.loop(0, nz)
        def _zero(i):
            pltpu.sync_copy(zbuf, accum.at[pl.ds(row0 + i * 16, 16)])

        plsc.subcore_barrier()

        def idxcopy(i, b):
            pltpu.sync_copy(eidx_hbm.at[w, i], ebuf[b])

        def gather(b):
            return pltpu.async_copy(x_hbm.at[ebuf[b].at[0]], rows[b],
                                    semg[b])

        def scatter(b):
            return pltpu.async_copy(rows[b], accum.at[ebuf[b].at[1]],
                                    sems[b], add=True)

        def wait_gather(b):
            pltpu.make_async_copy(x_hbm.at[ebuf[b].at[0]], rows[b],
                                  semg[b]).wait()

        def wait_scatter(b):
            pltpu.make_async_copy(rows[b], accum.at[ebuf[b].at[1]],
                                  sems[b]).wait()

        idxcopy(0, 0)
        gather(0)
        idxcopy(1, 1)
        gather(1)

        wait_gather(0)
        scatter(0)
        idxcopy(2, 2)
        gather(2)

        @pl.loop(0, (N_ITERS - 2) // 3)
        def _steps(g):
            for b_off in range(3):
                i = 3 * g + 1 + b_off
                b = (1 + b_off) % 3
                nb = b_off % 3
                wait_gather(b)
                scatter(b)

                @pl.when(i < N_ITERS - 2)
                def _prefetch():
                    wait_scatter(nb)
                    idxcopy(i + 2, nb)
                    gather(nb)

        wait_gather(1)
        scatter(1)

        wait_scatter(2)
        wait_scatter(0)
        wait_scatter(1)

        plsc.subcore_barrier()

        out_row0 = c * N_NODES + row0
        pltpu.sync_copy(accum.at[pl.ds(row0, ROWS_PER_TILE)],
                        out_hbm.at[pl.ds(out_row0, ROWS_PER_TILE)])

        @pl.when(s == NS - 1)
        def _tail():
            pltpu.sync_copy(accum.at[pl.ds(NS * ROWS_PER_TILE, 16)],
                            out_hbm.at[pl.ds(c * N_NODES + NS * ROWS_PER_TILE,
                                             16)])

    return sc_kernel


_sc_kernel = _make_sc_kernel()


def _combine_body(a_ref, b_ref, o_ref):
    o_ref[...] = a_ref[...] + b_ref[...]


@jax.jit
def _combine(partials):
    blk = 1000
    grid = N_NODES // blk
    return pl.pallas_call(
        _combine_body,
        grid=(grid,),
        in_specs=[
            pl.BlockSpec((blk, D), lambda i: (i, 0)),
            pl.BlockSpec((blk, D), lambda i: (i + N_NODES // blk, 0)),
        ],
        out_specs=pl.BlockSpec((blk, D), lambda i: (i, 0)),
        out_shape=jax.ShapeDtypeStruct((N_NODES, D), jnp.float32),
    )(partials, partials)


@jax.jit
def kernel(x, edge_index):
    eidx = edge_index.reshape(2, NC * NS, N_ITERS, CHUNK).transpose(1, 2, 0, 3)
    partials = _sc_kernel(x, eidx)
    return _combine(partials)
